# baseline (speedup 1.0000x reference)
import jax
import jax.numpy as jnp
from jax.experimental import pallas as pl


def kernel(data, mask_array):
    B, N, D = data.shape
    mask2 = mask_array.astype(jnp.int32).reshape(N, 1)
    Nb = 1024

    def body(d_ref, m_ref, o_ref):
        tile = jnp.concatenate([d_ref[b] for b in range(B)], axis=-1)
        o_ref[...] = jnp.where(m_ref[...] != 0, tile, 0.0)

    out2 = pl.pallas_call(
        body,
        grid=(N // Nb,),
        in_specs=[
            pl.BlockSpec((B, Nb, D), lambda i: (0, i, 0)),
            pl.BlockSpec((Nb, 1), lambda i: (i, 0)),
        ],
        out_specs=pl.BlockSpec((Nb, B * D), lambda i: (i, 0)),
        out_shape=jax.ShapeDtypeStruct((N, B * D), jnp.float32),
    )(data, mask2)
    return out2.reshape(N, B, D)
